# baseline (speedup 1.0000x reference)
import functools

import jax
import jax.numpy as jnp
from jax import lax
from jax.experimental import pallas as pl
from jax.experimental.pallas import tpu as pltpu
from jax.experimental.pallas import tpu_sc as plsc

B = 4096
L = 200
EMB = 64
PAIR = 2 * EMB
SCALE = 8.0

NC = 2
NS = 16
NW = NC * NS
TOK_PER_W = B * L // NW

LANES = 16
VPR = EMB // LANES

GW = 128
CHUNKS = TOK_PER_W // GW
NGRP = GW // LANES


def _body(idx2_hbm, par_hbm, table_hbm, pos_hbm, out_hbm,
          idx2_all, par_all, rows, outb, pos_v, sem_g, sem_o):
    wid = lax.axis_index("s") * NC + lax.axis_index("c")

    pltpu.sync_copy(idx2_hbm.at[wid], idx2_all)
    pltpu.sync_copy(par_hbm.at[wid], par_all)
    pltpu.sync_copy(pos_hbm, pos_v)

    out_base = wid * TOK_PER_W

    def start_gather(g, b):
        pltpu.async_copy(
            table_hbm.at[idx2_all.at[g]], rows.at[b], sem_g.at[b])

    def wait_gather(g, b):
        pltpu.make_async_copy(
            table_hbm.at[idx2_all.at[g]], rows.at[b], sem_g.at[b]).wait()

    def start_out(g, b):
        pltpu.async_copy(
            outb.at[b], out_hbm.at[pl.ds(out_base + g * GW, GW)],
            sem_o.at[b])

    def wait_out(b):
        pltpu.make_async_copy(
            outb.at[b], out_hbm.at[pl.ds(out_base, GW)], sem_o.at[b]).wait()

    start_gather(0, 0)

    def step(i, carry):
        for b in (0, 1):
            g = 2 * i + b

            @pl.when(g >= 2)
            def _():
                wait_out(b)

            @pl.when(g + 1 < CHUNKS)
            def _():
                start_gather(g + 1, 1 - b)

            wait_gather(g, b)

            off = lax.rem(g * GW, L)

            def grp_body(grp, c2):
                r0 = grp * LANES
                par16 = par_all[g, pl.ds(r0, LANES)]
                for k in range(LANES):
                    r = r0 + k
                    pc = par16[k]
                    p = off + r
                    p = p - jnp.where(p >= L, L, 0)
                    ph = p // 2
                    pp = (p % 2) * EMB
                    for j in range(VPR):
                        outb[b, r, pl.ds(j * LANES, LANES)] = (
                            rows[b, r, pl.ds(pc + j * LANES, LANES)] * SCALE
                            + pos_v[ph, pl.ds(pp + j * LANES, LANES)])
                return c2

            lax.fori_loop(0, NGRP, grp_body, 0)
            start_out(g, b)
        return carry

    lax.fori_loop(0, CHUNKS // 2, step, 0)
    wait_out(0)
    wait_out(1)


@functools.lru_cache(maxsize=1)
def _build():
    mesh = plsc.VectorSubcoreMesh(core_axis_name="c", subcore_axis_name="s")
    return pl.kernel(
        _body,
        mesh=mesh,
        compiler_params=pltpu.CompilerParams(disable_bounds_checks=True),
        out_type=jax.ShapeDtypeStruct((B * L, EMB), jnp.float32),
        scratch_types=[
            pltpu.VMEM((CHUNKS, GW), jnp.int32),
            pltpu.VMEM((CHUNKS, GW), jnp.int32),
            pltpu.VMEM((2, GW, PAIR), jnp.float32),
            pltpu.VMEM((2, GW, EMB), jnp.float32),
            pltpu.VMEM((L // 2, PAIR), jnp.float32),
            pltpu.SemaphoreType.DMA((2,)),
            pltpu.SemaphoreType.DMA((2,)),
        ],
    )


def kernel(tokens, table, pos_embedding):
    tok = tokens.reshape(-1).astype(jnp.int32).reshape(NW, CHUNKS, GW)
    idx2 = lax.shift_right_logical(tok, 1)
    par = jnp.bitwise_and(tok, 1) * EMB
    table2 = table.reshape(table.shape[0] // 2, PAIR)
    pos_p = pos_embedding[0, :L, :].reshape(L // 2, PAIR)
    out = _build()(idx2, par, table2, pos_p)
    return out.reshape(B, L, EMB)


# speedup vs baseline: 1.1169x; 1.1169x over previous
import functools

import jax
import jax.numpy as jnp
from jax import lax
from jax.experimental import pallas as pl
from jax.experimental.pallas import tpu as pltpu
from jax.experimental.pallas import tpu_sc as plsc

B = 4096
L = 200
EMB = 64
PADE = 128
SCALE = 8.0

NC = 2
NS = 16
NW = NC * NS
TOK_PER_W = B * L // NW

LANES = 16
VPR = EMB // LANES

GW = 128
CHUNKS = TOK_PER_W // GW


def _body(tokens_hbm, table_hbm, pos_hbm, out_hbm, idx_all, rows, outb, pos_v,
          sem_g, sem_o):
    wid = lax.axis_index("s") * NC + lax.axis_index("c")

    pltpu.sync_copy(tokens_hbm.at[wid], idx_all)
    pltpu.sync_copy(pos_hbm, pos_v)

    out_base = wid * TOK_PER_W

    def start_gather(g, b):
        pltpu.async_copy(
            table_hbm.at[idx_all.at[g]], rows.at[b], sem_g.at[b])

    def wait_gather(g, b):
        pltpu.make_async_copy(
            table_hbm.at[idx_all.at[g]], rows.at[b], sem_g.at[b]).wait()

    def start_out(g, b):
        pltpu.async_copy(
            outb.at[b], out_hbm.at[pl.ds(out_base + g * GW, GW)],
            sem_o.at[b])

    def wait_out(b):
        pltpu.make_async_copy(
            outb.at[b], out_hbm.at[pl.ds(out_base, GW)], sem_o.at[b]).wait()

    start_gather(0, 0)

    def step(i, carry):
        for b in (0, 1):
            g = 2 * i + b

            @pl.when(g >= 2)
            def _():
                wait_out(b)

            @pl.when(g + 1 < CHUNKS)
            def _():
                start_gather(g + 1, 1 - b)

            wait_gather(g, b)

            off = lax.rem(g * GW, L)

            def fma_row(r, c2):
                p = off + r
                p = p - jnp.where(p >= L, L, 0)
                ph = p // 2
                pc = (p % 2) * EMB
                for j in range(VPR):
                    outb[b, r, pl.ds(j * LANES, LANES)] = (
                        rows[b, r, pl.ds(j * LANES, LANES)] * SCALE
                        + pos_v[ph, pl.ds(pc + j * LANES, LANES)])
                return c2

            lax.fori_loop(0, GW, fma_row, 0, unroll=4)
            start_out(g, b)
        return carry

    lax.fori_loop(0, CHUNKS // 2, step, 0)
    wait_out(0)
    wait_out(1)


@functools.lru_cache(maxsize=1)
def _build():
    mesh = plsc.VectorSubcoreMesh(core_axis_name="c", subcore_axis_name="s")
    return pl.kernel(
        _body,
        mesh=mesh,
        out_type=jax.ShapeDtypeStruct((B * L, EMB), jnp.float32),
        scratch_types=[
            pltpu.VMEM((CHUNKS, GW), jnp.int32),
            pltpu.VMEM((2, GW, PADE), jnp.float32),
            pltpu.VMEM((2, GW, EMB), jnp.float32),
            pltpu.VMEM((L // 2, 2 * EMB), jnp.float32),
            pltpu.SemaphoreType.DMA((2,)),
            pltpu.SemaphoreType.DMA((2,)),
        ],
    )


def kernel(tokens, table, pos_embedding):
    tokens_w = tokens.reshape(-1).astype(jnp.int32).reshape(NW, CHUNKS, GW)
    table_p = jnp.pad(table, ((0, 0), (0, PADE - EMB)))
    pos_p = pos_embedding[0, :L, :].reshape(L // 2, 2 * EMB)
    out = _build()(tokens_w, table_p, pos_p)
    return out.reshape(B, L, EMB)
